# baseline (speedup 1.0000x reference)
import jax
import jax.numpy as jnp
from jax.experimental import pallas as pl

NUM_GROUPS = 2
TOKENS_PER_GROUP = 4096
HIDDEN_DIM = 4096
NUM_EXPERTS = 64

BLOCK_T = 512


def _router_block(x_ref, w_ref, b_ref, probs_ref, logits_ref, zacc_ref):
    i = pl.program_id(0)
    x = x_ref[...]
    w = w_ref[...]
    logits = jax.lax.dot_general(
        x, w, (((1,), (1,)), ((), ())), preferred_element_type=jnp.float32
    ) + b_ref[...]
    m = jnp.max(logits, axis=-1, keepdims=True)
    e = jnp.exp(logits - m)
    s = jnp.sum(e, axis=-1, keepdims=True)
    logits_ref[...] = logits
    probs_ref[...] = e / s
    log_z = m + jnp.log(s)
    partial = jnp.sum(log_z * log_z).reshape(1, 1)

    @pl.when(i == 0)
    def _init():
        zacc_ref[...] = jnp.zeros((1, 1), jnp.float32)

    zacc_ref[...] += partial


def kernel(token_inputs, W, b, expert_capacity):
    del expert_capacity
    total_tokens = NUM_GROUPS * TOKENS_PER_GROUP
    x = token_inputs.reshape(total_tokens, HIDDEN_DIM).astype(jnp.float32)
    b2 = b.reshape(1, NUM_EXPERTS).astype(jnp.float32)
    grid = (total_tokens // BLOCK_T,)

    probs, logits, zacc = pl.pallas_call(
        _router_block,
        grid=grid,
        in_specs=[
            pl.BlockSpec((BLOCK_T, HIDDEN_DIM), lambda i: (i, 0)),
            pl.BlockSpec((NUM_EXPERTS, HIDDEN_DIM), lambda i: (0, 0)),
            pl.BlockSpec((1, NUM_EXPERTS), lambda i: (0, 0)),
        ],
        out_specs=[
            pl.BlockSpec((BLOCK_T, NUM_EXPERTS), lambda i: (i, 0)),
            pl.BlockSpec((BLOCK_T, NUM_EXPERTS), lambda i: (i, 0)),
            pl.BlockSpec((1, 1), lambda i: (0, 0)),
        ],
        out_shape=[
            jax.ShapeDtypeStruct((total_tokens, NUM_EXPERTS), jnp.float32),
            jax.ShapeDtypeStruct((total_tokens, NUM_EXPERTS), jnp.float32),
            jax.ShapeDtypeStruct((1, 1), jnp.float32),
        ],
    )(x, W.astype(jnp.float32), b2)

    router_probs = probs.reshape(NUM_GROUPS, TOKENS_PER_GROUP, NUM_EXPERTS)
    router_logits = logits.reshape(NUM_GROUPS, TOKENS_PER_GROUP, NUM_EXPERTS)
    router_z_loss = zacc[0, 0] / total_tokens
    return (router_probs, router_logits, router_z_loss)
